# baseline (speedup 1.0000x reference)
import jax
import jax.numpy as jnp
from jax.experimental import pallas as pl
from jax.experimental.pallas import tpu as pltpu

W = 50
WD = 256
NBUF = 16
CP_ROWS = 2000
NCPS = 4


def _body(gath_ref, off_ref, sel_ref, sh_ref, scat_ref,
          cf, tail, pvm_hbm, w_ref, x_ref, y_ref, lw_ref, out_hbm,
          buf, pvstage, sem_win, sem_wb, sem_lw, sem_cpi, sem_cpo, sem_s):
    F, C, T = cf.shape
    B = gath_ref.shape[0]
    n_cp = T // CP_ROWS
    b = pl.program_id(0)
    slot = jax.lax.rem(b, NBUF)

    def win_start(i, s):
        for f in range(F):
            @pl.when(sel_ref[i] == 0)
            def _():
                pltpu.make_async_copy(
                    cf.at[f, :, pl.ds(off_ref[i] * 128, 128)],
                    buf.at[s, f, :, pl.ds(0, 128)], sem_win.at[s]).start()

            @pl.when(sel_ref[i] == 1)
            def _():
                pltpu.make_async_copy(
                    tail.at[f, :, pl.ds(0, 128)],
                    buf.at[s, f, :, pl.ds(0, 128)], sem_win.at[s]).start()

        @pl.when(sh_ref[i] + (W + 1) > 128)
        def _():
            for f in range(F):
                @pl.when(sel_ref[i] == 0)
                def _():
                    pltpu.make_async_copy(
                        cf.at[f, :, pl.ds(off_ref[i] * 128 + 128, 128)],
                        buf.at[s, f, :, pl.ds(128, 128)], sem_wb.at[s]).start()

                @pl.when(sel_ref[i] == 1)
                def _():
                    pltpu.make_async_copy(
                        tail.at[f, :, pl.ds(128, 128)],
                        buf.at[s, f, :, pl.ds(128, 128)], sem_wb.at[s]).start()

    def win_wait(i, s):
        for f in range(F):
            pltpu.make_async_copy(
                tail.at[f, :, pl.ds(0, 128)],
                buf.at[s, f, :, pl.ds(0, 128)], sem_win.at[s]).wait()

        @pl.when(sh_ref[i] + (W + 1) > 128)
        def _():
            for f in range(F):
                pltpu.make_async_copy(
                    tail.at[f, :, pl.ds(128, 128)],
                    buf.at[s, f, :, pl.ds(128, 128)], sem_wb.at[s]).wait()

    @pl.when(b == 0)
    def _():
        for i in range(NBUF - 1):
            win_start(i, i)

    pltpu.make_async_copy(
        pvm_hbm.at[pl.ds(gath_ref[b], 1)],
        lw_ref.at[pl.ds(b, 1)], sem_lw).start()

    def cp_in(i):
        return pltpu.make_async_copy(
            pvm_hbm.at[pl.ds(i * CP_ROWS, CP_ROWS)],
            pvstage.at[jax.lax.rem(i, NCPS)], sem_cpi.at[jax.lax.rem(i, NCPS)])

    def cp_out(i):
        return pltpu.make_async_copy(
            pvstage.at[jax.lax.rem(i, NCPS)],
            out_hbm.at[pl.ds(i * CP_ROWS, CP_ROWS)], sem_cpo.at[jax.lax.rem(i, NCPS)])

    @pl.when(jnp.logical_and(b >= NCPS, b < n_cp + NCPS))
    def _():
        cp_out(b - NCPS).wait()

    @pl.when(b < n_cp)
    def _():
        cp_in(b).start()

    @pl.when(jnp.logical_and(b >= 1, b <= n_cp))
    def _():
        cp_in(b - 1).wait()
        cp_out(b - 1).start()

    @pl.when(b + NBUF - 1 < B)
    def _():
        win_start(b + NBUF - 1, jax.lax.rem(b + NBUF - 1, NBUF))

    win_wait(b, slot)

    sh = sh_ref[b]
    ra = pltpu.roll(buf[slot, :, :, 0:128], -sh, 2)
    rb = pltpu.roll(buf[slot, :, :, 128:256], -sh, 2)
    lane = jax.lax.broadcasted_iota(jnp.int32, (F, C, 128), 2)
    wv = jnp.where(lane + sh < 128, ra, rb)
    x_ref[0] = wv
    y_ref[0] = wv[:, :, W] / wv[0:1, :, W - 1]

    @pl.when(b == B - 1)
    def _():
        for b2 in range(B):
            pltpu.make_async_copy(
                pvm_hbm.at[pl.ds(gath_ref[b2], 1)],
                lw_ref.at[pl.ds(b2, 1)], sem_lw).wait()
        for b2 in range(B):
            pltpu.make_async_copy(
                w_ref.at[pl.ds(b2, 1)],
                out_hbm.at[pl.ds(scat_ref[b2], 1)], sem_s).start()
        for b2 in range(B):
            pltpu.make_async_copy(
                w_ref.at[pl.ds(b2, 1)],
                out_hbm.at[pl.ds(scat_ref[b2], 1)], sem_s).wait()
        fix = pltpu.make_async_copy(
            pvm_hbm.at[pl.ds(T - 1, 1)], out_hbm.at[pl.ds(T - 1, 1)], sem_lw)
        fix.start()
        fix.wait()


def kernel(coin_features, pvm, index, w):
    F, C, T = coin_features.shape
    B = index.shape[0]
    idx = index.astype(jnp.int32)

    main_max_t0 = (T - WD) // 128
    tail_start = 128 * (main_max_t0 + 1)
    tail_len = T - tail_start
    t0 = idx // 128
    sel = (t0 > main_max_t0).astype(jnp.int32)
    off = jnp.where(sel == 1, 0, t0).astype(jnp.int32)
    sh = jnp.remainder(idx, 128).astype(jnp.int32)
    tail = jnp.pad(
        jax.lax.slice_in_dim(coin_features, tail_start, T, axis=2),
        ((0, 0), (0, 0), (0, WD - tail_len)))

    eq = idx[None, :] == idx[:, None]
    later_dup = jnp.triu(eq, 1).any(axis=1)
    scat = jnp.where(later_dup, T - 1, idx).astype(jnp.int32)
    gath = (idx - 1).astype(jnp.int32)

    grid_spec = pltpu.PrefetchScalarGridSpec(
        num_scalar_prefetch=5,
        grid=(B,),
        in_specs=[
            pl.BlockSpec(memory_space=pl.ANY),
            pl.BlockSpec(memory_space=pl.ANY),
            pl.BlockSpec(memory_space=pl.ANY),
            pl.BlockSpec((B, C), lambda b, *_: (0, 0)),
        ],
        out_specs=[
            pl.BlockSpec((1, F, C, 128), lambda b, *_: (b, 0, 0, 0)),
            pl.BlockSpec((1, F, C), lambda b, *_: (b, 0, 0)),
            pl.BlockSpec((B, C), lambda b, *_: (0, 0)),
            pl.BlockSpec(memory_space=pl.ANY),
        ],
        scratch_shapes=[
            pltpu.VMEM((NBUF, F, C, WD), jnp.float32),
            pltpu.VMEM((NCPS, CP_ROWS, C), jnp.float32),
            pltpu.SemaphoreType.DMA((NBUF,)),
            pltpu.SemaphoreType.DMA((NBUF,)),
            pltpu.SemaphoreType.DMA,
            pltpu.SemaphoreType.DMA((NCPS,)),
            pltpu.SemaphoreType.DMA((NCPS,)),
            pltpu.SemaphoreType.DMA,
        ],
    )
    x, y, last_w, pvm_new = pl.pallas_call(
        _body,
        grid_spec=grid_spec,
        out_shape=[
            jax.ShapeDtypeStruct((B, F, C, 128), jnp.float32),
            jax.ShapeDtypeStruct((B, F, C), jnp.float32),
            jax.ShapeDtypeStruct((B, C), jnp.float32),
            jax.ShapeDtypeStruct((T, C), jnp.float32),
        ],
        compiler_params=pltpu.CompilerParams(
            dimension_semantics=("arbitrary",),
        ),
    )(gath, off, sel, sh, scat, coin_features, tail, pvm, w)
    return x[:, :, :, :W], y, last_w, pvm_new


# speedup vs baseline: 1.0046x; 1.0046x over previous
import jax
import jax.numpy as jnp
from jax.experimental import pallas as pl
from jax.experimental.pallas import tpu as pltpu

W = 50
WD = 256
NBUF = 8
CP_ROWS = 2000
NCPS = 4


def _body(gath_ref, off_ref, sel_ref, sh_ref, scat_ref,
          cf, tail, pvm_hbm, w_ref, x_ref, y_ref, lw_ref, out_hbm,
          buf, pvstage, sem_win, sem_wb, sem_lw, sem_cpi, sem_cpo, sem_s):
    F, C, T = cf.shape
    B = gath_ref.shape[0]
    n_cp = T // CP_ROWS
    b = pl.program_id(0)
    slot = jax.lax.rem(b, NBUF)

    def win_start(i, s):
        @pl.when(sel_ref[i] == 0)
        def _():
            pltpu.make_async_copy(
                cf.at[:, :, pl.ds(off_ref[i] * 128, 128)],
                buf.at[s, :, :, pl.ds(0, 128)], sem_win.at[s]).start()

        @pl.when(sel_ref[i] == 1)
        def _():
            pltpu.make_async_copy(
                tail.at[:, :, pl.ds(0, 128)],
                buf.at[s, :, :, pl.ds(0, 128)], sem_win.at[s]).start()

        @pl.when(sh_ref[i] + (W + 1) > 128)
        def _():
            @pl.when(sel_ref[i] == 0)
            def _():
                pltpu.make_async_copy(
                    cf.at[:, :, pl.ds(off_ref[i] * 128 + 128, 128)],
                    buf.at[s, :, :, pl.ds(128, 128)], sem_wb.at[s]).start()

            @pl.when(sel_ref[i] == 1)
            def _():
                pltpu.make_async_copy(
                    tail.at[:, :, pl.ds(128, 128)],
                    buf.at[s, :, :, pl.ds(128, 128)], sem_wb.at[s]).start()

    def win_wait(i, s):
        pltpu.make_async_copy(
            tail.at[:, :, pl.ds(0, 128)],
            buf.at[s, :, :, pl.ds(0, 128)], sem_win.at[s]).wait()

        @pl.when(sh_ref[i] + (W + 1) > 128)
        def _():
            pltpu.make_async_copy(
                tail.at[:, :, pl.ds(128, 128)],
                buf.at[s, :, :, pl.ds(128, 128)], sem_wb.at[s]).wait()

    @pl.when(b == 0)
    def _():
        for i in range(NBUF - 1):
            win_start(i, i)

    pltpu.make_async_copy(
        pvm_hbm.at[pl.ds(gath_ref[b], 1)],
        lw_ref.at[pl.ds(b, 1)], sem_lw).start()

    def cp_in(i):
        return pltpu.make_async_copy(
            pvm_hbm.at[pl.ds(i * CP_ROWS, CP_ROWS)],
            pvstage.at[jax.lax.rem(i, NCPS)], sem_cpi.at[jax.lax.rem(i, NCPS)])

    def cp_out(i):
        return pltpu.make_async_copy(
            pvstage.at[jax.lax.rem(i, NCPS)],
            out_hbm.at[pl.ds(i * CP_ROWS, CP_ROWS)], sem_cpo.at[jax.lax.rem(i, NCPS)])

    @pl.when(jnp.logical_and(b >= NCPS, b < n_cp + NCPS))
    def _():
        cp_out(b - NCPS).wait()

    @pl.when(b < n_cp)
    def _():
        cp_in(b).start()

    @pl.when(jnp.logical_and(b >= 1, b <= n_cp))
    def _():
        cp_in(b - 1).wait()
        cp_out(b - 1).start()

    @pl.when(b + NBUF - 1 < B)
    def _():
        win_start(b + NBUF - 1, jax.lax.rem(b + NBUF - 1, NBUF))

    win_wait(b, slot)

    sh = sh_ref[b]
    ra = pltpu.roll(buf[slot, :, :, 0:128], -sh, 2)
    rb = pltpu.roll(buf[slot, :, :, 128:256], -sh, 2)
    lane = jax.lax.broadcasted_iota(jnp.int32, (F, C, 128), 2)
    wv = jnp.where(lane + sh < 128, ra, rb)
    x_ref[0] = wv
    y_ref[0] = wv[:, :, W] / wv[0:1, :, W - 1]

    @pl.when(b == B - 1)
    def _():
        for b2 in range(B):
            pltpu.make_async_copy(
                pvm_hbm.at[pl.ds(gath_ref[b2], 1)],
                lw_ref.at[pl.ds(b2, 1)], sem_lw).wait()
        for b2 in range(B):
            pltpu.make_async_copy(
                w_ref.at[pl.ds(b2, 1)],
                out_hbm.at[pl.ds(scat_ref[b2], 1)], sem_s).start()
        for b2 in range(B):
            pltpu.make_async_copy(
                w_ref.at[pl.ds(b2, 1)],
                out_hbm.at[pl.ds(scat_ref[b2], 1)], sem_s).wait()
        fix = pltpu.make_async_copy(
            pvm_hbm.at[pl.ds(T - 1, 1)], out_hbm.at[pl.ds(T - 1, 1)], sem_lw)
        fix.start()
        fix.wait()


def kernel(coin_features, pvm, index, w):
    F, C, T = coin_features.shape
    B = index.shape[0]
    idx = index.astype(jnp.int32)

    main_max_t0 = (T - WD) // 128
    tail_start = 128 * (main_max_t0 + 1)
    tail_len = T - tail_start
    t0 = idx // 128
    sel = (t0 > main_max_t0).astype(jnp.int32)
    off = jnp.where(sel == 1, 0, t0).astype(jnp.int32)
    sh = jnp.remainder(idx, 128).astype(jnp.int32)
    tail = jnp.pad(
        jax.lax.slice_in_dim(coin_features, tail_start, T, axis=2),
        ((0, 0), (0, 0), (0, WD - tail_len)))

    eq = idx[None, :] == idx[:, None]
    later_dup = jnp.triu(eq, 1).any(axis=1)
    scat = jnp.where(later_dup, T - 1, idx).astype(jnp.int32)
    gath = (idx - 1).astype(jnp.int32)

    grid_spec = pltpu.PrefetchScalarGridSpec(
        num_scalar_prefetch=5,
        grid=(B,),
        in_specs=[
            pl.BlockSpec(memory_space=pl.ANY),
            pl.BlockSpec(memory_space=pl.ANY),
            pl.BlockSpec(memory_space=pl.ANY),
            pl.BlockSpec((B, C), lambda b, *_: (0, 0)),
        ],
        out_specs=[
            pl.BlockSpec((1, F, C, 128), lambda b, *_: (b, 0, 0, 0)),
            pl.BlockSpec((1, F, C), lambda b, *_: (b, 0, 0)),
            pl.BlockSpec((B, C), lambda b, *_: (0, 0)),
            pl.BlockSpec(memory_space=pl.ANY),
        ],
        scratch_shapes=[
            pltpu.VMEM((NBUF, F, C, WD), jnp.float32),
            pltpu.VMEM((NCPS, CP_ROWS, C), jnp.float32),
            pltpu.SemaphoreType.DMA((NBUF,)),
            pltpu.SemaphoreType.DMA((NBUF,)),
            pltpu.SemaphoreType.DMA,
            pltpu.SemaphoreType.DMA((NCPS,)),
            pltpu.SemaphoreType.DMA((NCPS,)),
            pltpu.SemaphoreType.DMA,
        ],
    )
    x, y, last_w, pvm_new = pl.pallas_call(
        _body,
        grid_spec=grid_spec,
        out_shape=[
            jax.ShapeDtypeStruct((B, F, C, 128), jnp.float32),
            jax.ShapeDtypeStruct((B, F, C), jnp.float32),
            jax.ShapeDtypeStruct((B, C), jnp.float32),
            jax.ShapeDtypeStruct((T, C), jnp.float32),
        ],
        compiler_params=pltpu.CompilerParams(
            dimension_semantics=("arbitrary",),
        ),
    )(gath, off, sel, sh, scat, coin_features, tail, pvm, w)
    return x[:, :, :, :W], y, last_w, pvm_new
